# baseline (speedup 1.0000x reference)
import functools

import jax
import jax.numpy as jnp
import numpy as np
from jax import lax
from jax.experimental import pallas as pl
from jax.experimental.pallas import tpu as pltpu
from jax.experimental.pallas import tpu_sc as plsc

_D = 64
_NL = 8
_CHUNK = 128
_STEP = np.float32(2.0 / 6.0)


def _make_rfsq(nb, nseq):
    rows = nb * nseq
    info = plsc.get_sparse_core_info()
    nc, ns, lanes = info.num_cores, info.num_subcores, info.num_lanes
    nw = nc * ns
    rows_per_w = rows // nw
    nchunks = rows_per_w // _CHUNK
    groups = _CHUNK // lanes
    chunks_per_b = nseq // _CHUNK
    mesh = plsc.VectorSubcoreMesh(core_axis_name="c", subcore_axis_name="s")

    @functools.partial(
        pl.kernel,
        mesh=mesh,
        compiler_params=pltpu.CompilerParams(needs_layout_passes=False),
        out_type=[
            jax.ShapeDtypeStruct((nb, _D, nseq), jnp.float32),
            jax.ShapeDtypeStruct((nb, _D, _NL, nseq), jnp.int32),
        ],
        scratch_types=[
            pltpu.VMEM((_CHUNK * _D,), jnp.float32),
            pltpu.VMEM((_D * 16,), jnp.float32),
            pltpu.VMEM((_D, _CHUNK), jnp.float32),
            pltpu.VMEM((_D, _NL, _CHUNK), jnp.int32),
        ],
    )
    def rfsq(z_hbm, qsum_hbm, codes_hbm, zbuf, rt, zT, cbufT):
        wid = lax.axis_index("s") * nc + lax.axis_index("c")
        chunk0 = wid * nchunks
        iota = lax.iota(jnp.int32, lanes)

        half = np.float32(0.5)
        three_half = np.float32(1.5)
        magic = np.int32(0x5F3759DF)
        zerov = jnp.zeros((lanes,), jnp.float32)
        zeros8 = (zerov,) * 8

        def chunk_body(ci, carry):
            cidx = chunk0 + ci
            b_idx = cidx // chunks_per_b
            s0 = (cidx % chunks_per_b) * _CHUNK
            pltpu.sync_copy(z_hbm.at[pl.ds(cidx * (_CHUNK * _D), _CHUNK * _D)], zbuf)

            def group_body(g, gcarry):
                rloc = g * lanes + iota
                rv64 = rloc * _D

                @plsc.parallel_loop(0, _D, step=4, unroll=4, carry=zeros8)
                def stats0(j, acc):
                    acc = list(acc)
                    for k in range(4):
                        v = plsc.load_gather(zbuf, [rv64 + (j + k)])
                        rt[pl.ds((j + k) * lanes, lanes)] = v
                        zT[j + k, pl.ds(g * lanes, lanes)] = v
                        acc[k] = acc[k] + v
                        acc[4 + k] = acc[4 + k] + v * v
                    return tuple(acc)

                def layer_params(stats):
                    mean = ((stats[0] + stats[1]) + (stats[2] + stats[3])) * np.float32(1.0 / _D)
                    msq = ((stats[4] + stats[5]) + (stats[6] + stats[7])) * np.float32(1.0 / _D)
                    var = (msq - mean * mean) * np.float32(_D / (_D - 1.0))
                    var = jnp.maximum(var, np.float32(1e-30))
                    bits = lax.bitcast_convert_type(var, jnp.int32)
                    bits = magic - (bits >> 1)
                    y = lax.bitcast_convert_type(bits, jnp.float32)
                    xh = var * half
                    y = y * (three_half - xh * y * y)
                    y = y * (three_half - xh * y * y)
                    y = y * (three_half - xh * y * y)
                    std = var * y + np.float32(1e-5)
                    inv3 = np.float32(3.0) / std
                    c2 = np.float32(3.5) - mean * inv3
                    u = _STEP * std
                    vshift = mean - std
                    return inv3, c2, u, vshift

                def quantize(r, inv3, c2, u, vshift):
                    p = r * inv3 + c2
                    ii = jnp.clip(p.astype(jnp.int32), 0, 6)
                    zq = ii.astype(jnp.float32) * u + vshift
                    zq_out = r + (zq - r)
                    rn = r - zq_out
                    return ii, rn

                def layer_loop(l, stats):
                    inv3, c2, u, vshift = layer_params(stats)

                    @plsc.parallel_loop(0, _D, step=4, unroll=4, carry=zeros8)
                    def nacc(j, acc):
                        acc = list(acc)
                        for k in range(4):
                            r = rt[pl.ds((j + k) * lanes, lanes)]
                            ii, rn = quantize(r, inv3, c2, u, vshift)
                            cbufT[j + k, l, pl.ds(g * lanes, lanes)] = ii
                            rt[pl.ds((j + k) * lanes, lanes)] = rn
                            acc[k] = acc[k] + rn
                            acc[4 + k] = acc[4 + k] + rn * rn
                        return tuple(acc)

                    return nacc

                stats = stats0
                for l_static in range(_NL - 1):
                    stats = layer_loop(l_static, stats)

                inv3, c2, u, vshift = layer_params(stats)

                @plsc.parallel_loop(0, _D, step=4, unroll=4)
                def final_layer(j):
                    for k in range(4):
                        r = rt[pl.ds((j + k) * lanes, lanes)]
                        ii, rn = quantize(r, inv3, c2, u, vshift)
                        cbufT[j + k, _NL - 1, pl.ds(g * lanes, lanes)] = ii
                        zv = zT[j + k, pl.ds(g * lanes, lanes)]
                        zT[j + k, pl.ds(g * lanes, lanes)] = zv - rn

                return gcarry

            lax.fori_loop(0, groups, group_body, 0)

            pltpu.sync_copy(zT, qsum_hbm.at[b_idx, :, pl.ds(s0, _CHUNK)])
            pltpu.sync_copy(cbufT, codes_hbm.at[b_idx, :, :, pl.ds(s0, _CHUNK)])
            return carry

        lax.fori_loop(0, nchunks, chunk_body, 0)

    return rfsq


def kernel(z):
    b, s, d = z.shape
    qsumT, codesT = _make_rfsq(b, s)(z.reshape(b * s * d))
    qsum = jnp.transpose(qsumT, (0, 2, 1))
    codes = jnp.transpose(codesT, (0, 3, 1, 2))
    return qsum, codes


# speedup vs baseline: 1.2573x; 1.2573x over previous
import functools

import jax
import jax.numpy as jnp
import numpy as np
from jax import lax
from jax.experimental import pallas as pl
from jax.experimental.pallas import tpu as pltpu
from jax.experimental.pallas import tpu_sc as plsc

_D = 64
_NL = 8
_CHUNK = 128
_STEP = np.float32(2.0 / 6.0)


def _make_rfsq(nb, nseq):
    rows = nb * nseq
    info = plsc.get_sparse_core_info()
    nc, ns, lanes = info.num_cores, info.num_subcores, info.num_lanes
    nw = nc * ns
    rows_per_w = rows // nw
    nchunks = rows_per_w // _CHUNK
    groups = _CHUNK // lanes
    chunks_per_b = nseq // _CHUNK
    mesh = plsc.VectorSubcoreMesh(core_axis_name="c", subcore_axis_name="s")

    @functools.partial(
        pl.kernel,
        mesh=mesh,
        compiler_params=pltpu.CompilerParams(needs_layout_passes=False),
        out_type=[
            jax.ShapeDtypeStruct((nb, _D, nseq), jnp.float32),
            jax.ShapeDtypeStruct((nb, _D, _NL, nseq), jnp.int32),
        ],
        scratch_types=[
            pltpu.VMEM((_CHUNK * _D,), jnp.float32),
            pltpu.VMEM((_D * 16,), jnp.float32),
            pltpu.VMEM((_D, _CHUNK), jnp.float32),
            pltpu.VMEM((_D, _NL, _CHUNK), jnp.int32),
        ],
    )
    def rfsq(z_hbm, qsum_hbm, codes_hbm, zbuf, rt, zT, cbufT):
        wid = lax.axis_index("s") * nc + lax.axis_index("c")
        chunk0 = wid * nchunks
        iota = lax.iota(jnp.int32, lanes)

        half = np.float32(0.5)
        three_half = np.float32(1.5)
        magic = np.int32(0x5F3759DF)
        zerov = jnp.zeros((lanes,), jnp.float32)
        zeros8 = (zerov,) * 8

        def chunk_body(ci, carry):
            cidx = chunk0 + ci
            b_idx = cidx // chunks_per_b
            s0 = (cidx % chunks_per_b) * _CHUNK
            pltpu.sync_copy(z_hbm.at[pl.ds(cidx * (_CHUNK * _D), _CHUNK * _D)], zbuf)

            def group_body(g, gcarry):
                rloc = g * lanes + iota
                rv64 = rloc * _D

                @plsc.parallel_loop(0, _D, step=4, unroll=4, carry=zeros8)
                def stats0(j, acc):
                    acc = list(acc)
                    for k in range(4):
                        v = plsc.load_gather(zbuf, [rv64 + (j + k)])
                        rt[pl.ds((j + k) * lanes, lanes)] = v
                        zT[j + k, pl.ds(g * lanes, lanes)] = v
                        acc[k] = acc[k] + v
                        acc[4 + k] = acc[4 + k] + v * v
                    return tuple(acc)

                def layer_params(stats):
                    mean = ((stats[0] + stats[1]) + (stats[2] + stats[3])) * np.float32(1.0 / _D)
                    msq = ((stats[4] + stats[5]) + (stats[6] + stats[7])) * np.float32(1.0 / _D)
                    var = (msq - mean * mean) * np.float32(_D / (_D - 1.0))
                    var = jnp.maximum(var, np.float32(1e-30))
                    bits = lax.bitcast_convert_type(var, jnp.int32)
                    bits = magic - (bits >> 1)
                    y = lax.bitcast_convert_type(bits, jnp.float32)
                    xh = var * half
                    y = y * (three_half - xh * y * y)
                    y = y * (three_half - xh * y * y)
                    y = y * (three_half - xh * y * y)
                    std = var * y + np.float32(1e-5)
                    inv3 = np.float32(3.0) / std
                    c2 = np.float32(3.5) - mean * inv3
                    u = _STEP * std
                    vshift = mean - std
                    return inv3, c2, u, vshift

                def quantize(r, inv3, c2, u, vshift):
                    p = r * inv3 + c2
                    ii = jnp.clip(p.astype(jnp.int32), 0, 6)
                    zq = ii.astype(jnp.float32) * u + vshift
                    zq_out = r + (zq - r)
                    rn = r - zq_out
                    return ii, rn

                def layer_loop(l, stats):
                    inv3, c2, u, vshift = layer_params(stats)

                    @plsc.parallel_loop(0, _D, step=4, unroll=4, carry=zeros8)
                    def nacc(j, acc):
                        acc = list(acc)
                        for k in range(4):
                            r = rt[pl.ds((j + k) * lanes, lanes)]
                            ii, rn = quantize(r, inv3, c2, u, vshift)
                            cbufT[j + k, l, pl.ds(g * lanes, lanes)] = ii
                            rt[pl.ds((j + k) * lanes, lanes)] = rn
                            acc[k] = acc[k] + rn
                            acc[4 + k] = acc[4 + k] + rn * rn
                        return tuple(acc)

                    return nacc

                stats = lax.fori_loop(0, _NL - 1, layer_loop, stats0)

                inv3, c2, u, vshift = layer_params(stats)

                @plsc.parallel_loop(0, _D, step=4, unroll=4)
                def final_layer(j):
                    for k in range(4):
                        r = rt[pl.ds((j + k) * lanes, lanes)]
                        ii, rn = quantize(r, inv3, c2, u, vshift)
                        cbufT[j + k, _NL - 1, pl.ds(g * lanes, lanes)] = ii
                        zv = zT[j + k, pl.ds(g * lanes, lanes)]
                        zT[j + k, pl.ds(g * lanes, lanes)] = zv - rn

                return gcarry

            lax.fori_loop(0, groups, group_body, 0)

            pltpu.sync_copy(zT, qsum_hbm.at[b_idx, :, pl.ds(s0, _CHUNK)])
            pltpu.sync_copy(cbufT, codes_hbm.at[b_idx, :, :, pl.ds(s0, _CHUNK)])
            return carry

        lax.fori_loop(0, nchunks, chunk_body, 0)

    return rfsq


def kernel(z):
    b, s, d = z.shape
    qsumT, codesT = _make_rfsq(b, s)(z.reshape(b * s * d))
    qsum = jnp.transpose(qsumT, (0, 2, 1))
    codes = jnp.transpose(codesT, (0, 3, 1, 2))
    return qsum, codes


# speedup vs baseline: 1.3312x; 1.0588x over previous
import functools

import jax
import jax.numpy as jnp
import numpy as np
from jax import lax
from jax.experimental import pallas as pl
from jax.experimental.pallas import tpu as pltpu
from jax.experimental.pallas import tpu_sc as plsc

_D = 64
_NL = 8
_CHUNK = 128
_STEP = np.float32(2.0 / 6.0)


def _make_rfsq(nb, nseq):
    rows = nb * nseq
    info = plsc.get_sparse_core_info()
    nc, ns, lanes = info.num_cores, info.num_subcores, info.num_lanes
    nw = nc * ns
    rows_per_w = rows // nw
    nchunks = rows_per_w // _CHUNK
    groups = _CHUNK // lanes
    chunks_per_b = nseq // _CHUNK
    mesh = plsc.VectorSubcoreMesh(core_axis_name="c", subcore_axis_name="s")

    @functools.partial(
        pl.kernel,
        mesh=mesh,
        compiler_params=pltpu.CompilerParams(needs_layout_passes=False),
        out_type=[
            jax.ShapeDtypeStruct((nb, _D, nseq), jnp.float32),
            jax.ShapeDtypeStruct((nb, _D, _NL, nseq), jnp.int32),
        ],
        scratch_types=[
            pltpu.VMEM((_CHUNK * _D,), jnp.float32),
            pltpu.VMEM((_D * 16,), jnp.float32),
            pltpu.VMEM((_D, _CHUNK), jnp.float32),
            pltpu.VMEM((_D, _NL, _CHUNK), jnp.int32),
        ],
    )
    def rfsq(z_hbm, qsum_hbm, codes_hbm, zbuf, rt, zT, cbufT):
        wid = lax.axis_index("s") * nc + lax.axis_index("c")
        chunk0 = wid * nchunks
        iota = lax.iota(jnp.int32, lanes)

        half = np.float32(0.5)
        three_half = np.float32(1.5)
        magic = np.int32(0x5F3759DF)
        zerov = jnp.zeros((lanes,), jnp.float32)
        zeros8 = (zerov,) * 8

        def chunk_body(ci, carry):
            cidx = chunk0 + ci
            b_idx = cidx // chunks_per_b
            s0 = (cidx % chunks_per_b) * _CHUNK
            pltpu.sync_copy(z_hbm.at[pl.ds(cidx * (_CHUNK * _D), _CHUNK * _D)], zbuf)

            def group_body(g, gcarry):
                rloc = g * lanes + iota
                rv64 = rloc * _D

                @plsc.parallel_loop(0, _D, step=4, unroll=4, carry=zeros8)
                def stats0(j, acc):
                    acc = list(acc)
                    for k in range(4):
                        v = plsc.load_gather(zbuf, [rv64 + (j + k)])
                        rt[pl.ds((j + k) * lanes, lanes)] = v
                        zT[j + k, pl.ds(g * lanes, lanes)] = v
                        acc[k] = acc[k] + v
                        acc[4 + k] = acc[4 + k] + v * v
                    return tuple(acc)

                def layer_params(stats):
                    mean = ((stats[0] + stats[1]) + (stats[2] + stats[3])) * np.float32(1.0 / _D)
                    msq = ((stats[4] + stats[5]) + (stats[6] + stats[7])) * np.float32(1.0 / _D)
                    var = (msq - mean * mean) * np.float32(_D / (_D - 1.0))
                    var = jnp.maximum(var, np.float32(1e-30))
                    bits = lax.bitcast_convert_type(var, jnp.int32)
                    bits = magic - (bits >> 1)
                    y = lax.bitcast_convert_type(bits, jnp.float32)
                    xh = var * half
                    y = y * (three_half - xh * y * y)
                    y = y * (three_half - xh * y * y)
                    y = y * (three_half - xh * y * y)
                    std = var * y + np.float32(1e-5)
                    inv3 = np.float32(3.0) / std
                    c2 = np.float32(3.5) - mean * inv3
                    u = _STEP * std
                    vshift = mean - std
                    return inv3, c2, u, vshift

                def quantize(r, inv3, c2, u, vshift):
                    p = r * inv3 + c2
                    ii = jnp.clip(p.astype(jnp.int32), 0, 6)
                    zq = ii.astype(jnp.float32) * u + vshift
                    rn = r - zq
                    return ii, rn

                def layer_loop(l, stats):
                    inv3, c2, u, vshift = layer_params(stats)

                    @plsc.parallel_loop(0, _D, step=4, unroll=4, carry=zeros8)
                    def nacc(j, acc):
                        acc = list(acc)
                        for k in range(4):
                            r = rt[pl.ds((j + k) * lanes, lanes)]
                            ii, rn = quantize(r, inv3, c2, u, vshift)
                            cbufT[j + k, l, pl.ds(g * lanes, lanes)] = ii
                            rt[pl.ds((j + k) * lanes, lanes)] = rn
                            acc[k] = acc[k] + rn
                            acc[4 + k] = acc[4 + k] + rn * rn
                        return tuple(acc)

                    return nacc

                stats = lax.fori_loop(0, _NL - 1, layer_loop, stats0)

                inv3, c2, u, vshift = layer_params(stats)

                @plsc.parallel_loop(0, _D, step=4, unroll=4)
                def final_layer(j):
                    for k in range(4):
                        r = rt[pl.ds((j + k) * lanes, lanes)]
                        ii, rn = quantize(r, inv3, c2, u, vshift)
                        cbufT[j + k, _NL - 1, pl.ds(g * lanes, lanes)] = ii
                        zv = zT[j + k, pl.ds(g * lanes, lanes)]
                        zT[j + k, pl.ds(g * lanes, lanes)] = zv - rn

                return gcarry

            lax.fori_loop(0, groups, group_body, 0)

            pltpu.sync_copy(zT, qsum_hbm.at[b_idx, :, pl.ds(s0, _CHUNK)])
            pltpu.sync_copy(cbufT, codes_hbm.at[b_idx, :, :, pl.ds(s0, _CHUNK)])
            return carry

        lax.fori_loop(0, nchunks, chunk_body, 0)

    return rfsq


def kernel(z):
    b, s, d = z.shape
    qsumT, codesT = _make_rfsq(b, s)(z.reshape(b * s * d))
    qsum = jnp.transpose(qsumT, (0, 2, 1))
    codes = jnp.transpose(codesT, (0, 3, 1, 2))
    return qsum, codes
